# Initial kernel scaffold; baseline (speedup 1.0000x reference)
import jax
import jax.numpy as jnp
from jax.experimental import pallas as pl


def kernel(tokens, attention_mask, table):
    raise NotImplementedError("write your pallas kernel here")


# speedup vs baseline: 2.0811x; 2.0811x over previous
import functools

import jax
import jax.numpy as jnp
from jax import lax
from jax.experimental import pallas as pl
from jax.experimental.pallas import tpu as pltpu
from jax.experimental.pallas import tpu_sc as plsc


def _make_lookup(n_tokens: int, vocab: int, d: int):
    info = plsc.get_sparse_core_info()
    nw = info.num_cores * info.num_subcores
    assert n_tokens % (8 * nw) == 0
    n_per_w = n_tokens // nw
    chunk = 128
    while n_per_w % chunk:
        chunk //= 2
    n_chunks = n_per_w // chunk
    mesh = plsc.VectorSubcoreMesh(core_axis_name="c", subcore_axis_name="s")

    @functools.partial(
        pl.kernel,
        mesh=mesh,
        out_type=jax.ShapeDtypeStruct((n_tokens, d), jnp.float32),
        scratch_types=[
            pltpu.VMEM((n_per_w,), jnp.int32),
            pltpu.VMEM((chunk, d), jnp.float32),
            pltpu.SemaphoreType.DMA,
        ],
    )
    def lookup(table_hbm, idx_hbm, out_hbm, idx_v, rows_v, gsem):
        wid = lax.axis_index("s") * info.num_cores + lax.axis_index("c")
        base = wid * n_per_w
        pltpu.sync_copy(idx_hbm.at[pl.ds(base, n_per_w)], idx_v)

        def body(ci, _):
            off = ci * chunk
            pltpu.async_copy(
                table_hbm.at[idx_v.at[pl.ds(off, chunk)]], rows_v, gsem
            ).wait()
            pltpu.sync_copy(rows_v, out_hbm.at[pl.ds(base + off, chunk)])
            return 0

        lax.fori_loop(0, n_chunks, body, 0)

    return lookup


def kernel(tokens, attention_mask, table):
    b, s = tokens.shape
    vocab, d = table.shape
    idx = tokens.reshape(b * s).astype(jnp.int32)
    out = _make_lookup(b * s, vocab, d)(table, idx)
    return out.reshape(b, s, d)


# speedup vs baseline: 2.0875x; 1.0031x over previous
import functools

import jax
import jax.numpy as jnp
from jax import lax
from jax.experimental import pallas as pl
from jax.experimental.pallas import tpu as pltpu
from jax.experimental.pallas import tpu_sc as plsc


def _make_lookup(n_tokens: int, vocab: int, d: int):
    info = plsc.get_sparse_core_info()
    nw = info.num_cores * info.num_subcores
    assert n_tokens % (8 * nw) == 0
    n_per_w = n_tokens // nw
    chunk = 64
    while n_per_w % (2 * chunk):
        chunk //= 2
    n_chunks = n_per_w // chunk
    chunk_bytes = chunk * d * 4
    mesh = plsc.VectorSubcoreMesh(core_axis_name="c", subcore_axis_name="s")

    @functools.partial(
        pl.kernel,
        mesh=mesh,
        out_type=jax.ShapeDtypeStruct((n_tokens, d), jnp.float32),
        scratch_types=[
            pltpu.VMEM((n_per_w,), jnp.int32),
            pltpu.VMEM((chunk, d), jnp.float32),
            pltpu.VMEM((chunk, d), jnp.float32),
            pltpu.SemaphoreType.DMA,
            pltpu.SemaphoreType.DMA,
        ],
    )
    def lookup(table_hbm, idx_hbm, out_hbm, idx_v, rows0, rows1, sem0, sem1):
        wid = lax.axis_index("s") * info.num_cores + lax.axis_index("c")
        base = wid * n_per_w
        pltpu.sync_copy(idx_hbm.at[pl.ds(base, n_per_w)], idx_v)

        bufs = (rows0, rows1)
        sems = (sem0, sem1)

        def start_gather(ci, b):
            pltpu.async_copy(
                table_hbm.at[idx_v.at[pl.ds(ci * chunk, chunk)]], bufs[b], sems[b]
            )

        start_gather(0, 0)

        def body(g, _):
            c0 = g * 2
            for b in range(2):
                ci = c0 + b
                nxt = ci + 1

                @pl.when(nxt < n_chunks)
                def _():
                    start_gather(nxt, 1 - b)

                pltpu.make_async_copy(
                    table_hbm.at[pl.ds(0, chunk)], bufs[b], sems[b]
                ).wait()
                pltpu.sync_copy(
                    bufs[b], out_hbm.at[pl.ds(base + ci * chunk, chunk)]
                )
            return 0

        lax.fori_loop(0, n_chunks // 2, body, 0)

    return lookup


def kernel(tokens, attention_mask, table):
    b, s = tokens.shape
    vocab, d = table.shape
    idx = tokens.reshape(b * s).astype(jnp.int32)
    out = _make_lookup(b * s, vocab, d)(table, idx)
    return out.reshape(b, s, d)


# speedup vs baseline: 2.0894x; 1.0009x over previous
import functools

import jax
import jax.numpy as jnp
from jax import lax
from jax.experimental import pallas as pl
from jax.experimental.pallas import tpu as pltpu
from jax.experimental.pallas import tpu_sc as plsc


def _make_lookup(n_tokens: int, vocab: int, d: int):
    info = plsc.get_sparse_core_info()
    nw = info.num_cores * info.num_subcores
    assert n_tokens % (8 * nw) == 0
    n_per_w = n_tokens // nw
    chunk = 64
    while n_per_w % (2 * chunk):
        chunk //= 2
    n_chunks = n_per_w // chunk
    mesh = plsc.VectorSubcoreMesh(core_axis_name="c", subcore_axis_name="s")

    @functools.partial(
        pl.kernel,
        mesh=mesh,
        out_type=jax.ShapeDtypeStruct((n_tokens, d), jnp.float32),
        scratch_types=[
            pltpu.VMEM((n_per_w,), jnp.int32),
            pltpu.VMEM((chunk, d), jnp.float32),
            pltpu.VMEM((chunk, d), jnp.float32),
            pltpu.SemaphoreType.DMA,
            pltpu.SemaphoreType.DMA,
        ],
    )
    def lookup(table_hbm, idx_hbm, out_hbm, idx_v, rows0, rows1, sem0, sem1):
        wid = lax.axis_index("s") * info.num_cores + lax.axis_index("c")
        base = wid * n_per_w
        pltpu.sync_copy(idx_hbm.at[pl.ds(base, n_per_w)], idx_v)

        bufs = (rows0, rows1)
        sems = (sem0, sem1)

        def start_gather(ci, b):
            pltpu.async_copy(
                table_hbm.at[idx_v.at[pl.ds(ci * chunk, chunk)]], bufs[b], sems[b]
            )

        start_gather(0, 0)

        def body(g, _):
            c0 = g * 2
            for b in range(2):
                ci = c0 + b
                nxt = ci + 1

                @pl.when(nxt < n_chunks)
                def _():
                    start_gather(nxt, 1 - b)

                pltpu.make_async_copy(
                    table_hbm.at[pl.ds(0, chunk)], bufs[b], sems[b]
                ).wait()
                pltpu.sync_copy(
                    bufs[b], out_hbm.at[pl.ds(base + ci * chunk, chunk)]
                )
            return 0

        lax.fori_loop(0, n_chunks // 2, body, 0)

    return lookup


def kernel(tokens, attention_mask, table):
    b, s = tokens.shape
    vocab, d = table.shape
    vocab_pad = -(-vocab // 128) * 128
    if vocab_pad != vocab:
        table = jnp.pad(table, ((0, vocab_pad - vocab), (0, 0)))
    idx = tokens.reshape(b * s).astype(jnp.int32)
    out = _make_lookup(b * s, vocab_pad, d)(table, idx)
    return out.reshape(b, s, d)
